# Initial kernel scaffold; baseline (speedup 1.0000x reference)
import jax
import jax.numpy as jnp
from jax.experimental import pallas as pl


def kernel(obs, is_alive, W1, b1, Wrel1, brel1, Wroot1, W2, b2, Wrel2, brel2, Wroot2, W3, b3, Wrel3, brel3, Wroot3, Wl, bl, Wv1, bv1, Wv2, bv2):
    raise NotImplementedError("write your pallas kernel here")


# speedup vs baseline: 20.4191x; 20.4191x over previous
import functools

import jax
import jax.numpy as jnp
import numpy as np
from jax import lax
from jax.experimental import pallas as pl
from jax.experimental.pallas import tpu as pltpu

N = 10000
NP = 10240
TILE = 1024
GRID = NP // TILE
RB = 8
K1, K2, K3 = 5001, 2501, 1251
NEG = float(np.float32(-1e30))

_DISA1 = np.float32(2.0) ** np.float32(-0.5)
_DIS01 = np.float32(N + 1.0) ** np.float32(-0.5)
SELF1 = float(np.float32(_DISA1 * _DISA1))
CROSS1 = float(np.float32(_DISA1 * _DIS01))


def _kth_threshold(sarr, s_extra, k):
    b = lax.bitcast_convert_type(sarr, jnp.uint32)
    flip = jnp.where(b >> 31 == 1, jnp.uint32(0xFFFFFFFF), jnp.uint32(0x80000000))
    u = b ^ flip
    bx = lax.bitcast_convert_type(s_extra, jnp.uint32)
    ux = bx ^ jnp.where(bx >> 31 == 1, jnp.uint32(0xFFFFFFFF), jnp.uint32(0x80000000))

    def body(i, t):
        cand = t | (jnp.uint32(1) << (jnp.uint32(31) - jnp.uint32(i)))
        cnt = jnp.sum((u >= cand).astype(jnp.int32)) + (ux >= cand).astype(jnp.int32)
        return jnp.where(cnt >= k, cand, t)

    t = lax.fori_loop(0, 32, body, jnp.uint32(0))
    tb = jnp.where(t & jnp.uint32(0x80000000) != 0,
                   t ^ jnp.uint32(0x80000000), ~t)
    return lax.bitcast_convert_type(tb, jnp.float32)


def _k1_body(obs_ref, w1_ref, b1_ref, wroot1_ref,
             a_ref, r1_ref, hsum_ref, asum_ref):
    t = pl.program_id(0)
    h = jnp.dot(obs_ref[...], w1_ref[...], preferred_element_type=jnp.float32)
    rid = t * TILE + lax.broadcasted_iota(jnp.int32, (TILE, 1), 0)
    valid = rid < N
    a = jnp.maximum(h * SELF1 + b1_ref[...], 0.0)
    a = jnp.where(valid, a, 0.0)
    a_ref[...] = a
    r = lax.dot_general(a, wroot1_ref[...], (((1,), (1,)), ((), ())),
                        preferred_element_type=jnp.float32)
    r1_ref[...] = jnp.where(valid, r, NEG)

    @pl.when(t == 0)
    def _():
        hsum_ref[...] = jnp.zeros_like(hsum_ref)
        asum_ref[...] = jnp.zeros_like(asum_ref)

    hsum_ref[...] += jnp.sum(h, axis=0, keepdims=True)
    asum_ref[...] += jnp.sum(a, axis=0, keepdims=True)


def _mid_body(kcur, degc_base, a_ref, r_ref, w_ref, b_ref, wrootn_ref,
              sig_ref, t_ref, c_ref, hc_ref,
              x_ref, rn_ref, hsum_ref, xsum_ref, mx_ref, sm_ref):
    t = pl.program_id(0)
    c = c_ref[0, 0]
    s = sig_ref[0, 0] + r_ref[...]
    alive = s >= t_ref[0, 0]
    g = jnp.where(alive, jnp.tanh(s), 0.0)
    y = a_ref[...] * g
    h = jnp.dot(y, w_ref[...], preferred_element_type=jnp.float32)
    disc = lax.rsqrt(jnp.float32(degc_base) - c)
    disa = lax.rsqrt(1.0 + c)
    out = (c * (disc * disa)) * hc_ref[...] + h * (disa * disa) + b_ref[...]
    x = jnp.where(alive, jnp.maximum(out, 0.0), 0.0)
    x_ref[...] = x
    rn = lax.dot_general(x, wrootn_ref[...], (((1,), (1,)), ((), ())),
                         preferred_element_type=jnp.float32)
    rn_ref[...] = jnp.where(alive, rn, NEG)

    @pl.when(t == 0)
    def _():
        hsum_ref[...] = jnp.zeros_like(hsum_ref)
        xsum_ref[...] = jnp.zeros_like(xsum_ref)
        sm_ref[...] = jnp.zeros_like(sm_ref)
        mx_ref[...] = jnp.full_like(mx_ref, NEG)

    hsum_ref[...] += jnp.sum(h, axis=0, keepdims=True)
    xsum_ref[...] += jnp.sum(x, axis=0, keepdims=True)
    sm_ref[...] += jnp.sum(y, axis=0, keepdims=True)
    mx_ref[...] = jnp.maximum(mx_ref[...],
                              jnp.max(jnp.where(alive, y, NEG), axis=0,
                                      keepdims=True))


def _c1_body(kk, r_ref, hsum_ref, asum_ref, b1_ref, wrel_ref, brel_ref,
             wroot_ref, wnext_ref,
             t_ref, c_ref, sig_ref, yc_ref, hc_ref):
    xc = jnp.maximum(hsum_ref[...] * CROSS1 + b1_ref[...], 0.0)
    brel = brel_ref[0, 0]
    sig = jnp.sum(xc * wrel_ref[...]) + brel
    sc = jnp.sum(asum_ref[...] * wrel_ref[...]) + brel + jnp.sum(xc * wroot_ref[...])
    thr = _kth_threshold(sig + r_ref[...], sc, kk)
    c = (sc >= thr).astype(jnp.float32)
    yc = xc * (jnp.tanh(sc) * c)
    t_ref[...] = jnp.reshape(thr, (1, 1))
    c_ref[...] = jnp.reshape(c, (1, 1))
    sig_ref[...] = jnp.reshape(sig, (1, 1))
    yc_ref[...] = yc
    hc_ref[...] = jnp.dot(yc, wnext_ref[...], preferred_element_type=jnp.float32)


def _cn_body(kk, degc_base, r_ref, hsum_ref, xsum_ref, hc_ref, cprev_ref,
             b_ref, wrel_ref, brel_ref, wroot_ref, wnext_ref,
             t_ref, c_ref, sig_ref, yc_ref, hcn_ref):
    cp = cprev_ref[0, 0]
    disc = lax.rsqrt(jnp.float32(degc_base) - cp)
    disa = lax.rsqrt(1.0 + cp)
    xc = jnp.maximum((disc * disa) * hsum_ref[...] + (disc * disc) * hc_ref[...]
                     + b_ref[...], 0.0) * cp
    brel = brel_ref[0, 0]
    sig = cp * jnp.sum(xc * wrel_ref[...]) + brel
    sc_live = jnp.sum(xsum_ref[...] * wrel_ref[...]) + brel + jnp.sum(xc * wroot_ref[...])
    sc = jnp.where(cp > 0, sc_live, NEG)
    thr = _kth_threshold(sig + r_ref[...], sc, kk)
    c = (sc >= thr).astype(jnp.float32)
    yc = xc * (jnp.tanh(sc) * c)
    t_ref[...] = jnp.reshape(thr, (1, 1))
    c_ref[...] = jnp.reshape(c, (1, 1))
    sig_ref[...] = jnp.reshape(sig, (1, 1))
    yc_ref[...] = yc
    hcn_ref[...] = jnp.dot(yc, wnext_ref[...], preferred_element_type=jnp.float32)


def _k7_body(x_ref, r_ref, sig_ref, t_ref, mx_ref, sm_ref):
    t = pl.program_id(0)
    s = sig_ref[0, 0] + r_ref[...]
    alive = s >= t_ref[0, 0]
    y = x_ref[...] * jnp.where(alive, jnp.tanh(s), 0.0)

    @pl.when(t == 0)
    def _():
        sm_ref[...] = jnp.zeros_like(sm_ref)
        mx_ref[...] = jnp.full_like(mx_ref, NEG)

    sm_ref[...] += jnp.sum(y, axis=0, keepdims=True)
    mx_ref[...] = jnp.maximum(mx_ref[...],
                              jnp.max(jnp.where(alive, y, NEG), axis=0,
                                      keepdims=True))


def _k8_body(mx1_ref, sm1_ref, mx2_ref, sm2_ref, mx3_ref, sm3_ref,
             yc1_ref, yc2_ref, yc3_ref, c1_ref, c2_ref, c3_ref,
             wlt_ref, wlb_ref, bl_ref, wv1_ref, bv1_ref, wv2_ref, bv2_ref,
             v_ref):
    def merge(mx_ref, sm_ref, yc_ref, c_ref, k):
        c = c_ref[0, 0]
        mx = jnp.maximum(mx_ref[...], jnp.where(c > 0, yc_ref[...], NEG))
        mean = (sm_ref[...] + yc_ref[...]) / jnp.float32(k)
        return mx, mean

    m1, a1 = merge(mx1_ref, sm1_ref, yc1_ref, c1_ref, K1)
    m2, a2 = merge(mx2_ref, sm2_ref, yc2_ref, c2_ref, K2)
    m3, a3 = merge(mx3_ref, sm3_ref, yc3_ref, c3_ref, K3)
    mx = m1 + m2 + m3
    mn = a1 + a2 + a3
    state = (jnp.dot(mx, wlt_ref[...], preferred_element_type=jnp.float32)
             + jnp.dot(mn, wlb_ref[...], preferred_element_type=jnp.float32)
             + bl_ref[...])
    sv = jnp.maximum(jnp.dot(state, wv1_ref[...],
                             preferred_element_type=jnp.float32)
                     + bv1_ref[...], 0.0)
    v_ref[...] = jnp.reshape(jnp.sum(sv * wv2_ref[...]) + bv2_ref[0, 0], (1, 1))


def _vec(x):
    return jax.ShapeDtypeStruct((1, x), jnp.float32)


_SCAL = jax.ShapeDtypeStruct((1, 1), jnp.float32)
_FULL = lambda shape: pl.BlockSpec(shape, lambda t: (0, 0))
_ROWS = pl.BlockSpec((TILE, 512), lambda t: (t, 0))
_SCORES = pl.BlockSpec((TILE, 1), lambda t: (t, 0))


def kernel(obs, is_alive, W1, b1, Wrel1, brel1, Wroot1, W2, b2, Wrel2, brel2,
           Wroot2, W3, b3, Wrel3, brel3, Wroot3, Wl, bl, Wv1, bv1, Wv2, bv2):
    f32 = jnp.float32
    obs_p = jnp.pad(obs, ((0, NP - N), (0, 0)))
    row = lambda w: w.reshape(1, -1).astype(f32)
    b1r, b2r, b3r, blr, bv1r = row(b1), row(b2), row(b3), row(bl), row(bv1)
    wrel1r, wrel2r, wrel3r = row(Wrel1), row(Wrel2), row(Wrel3)
    wroot1r, wroot2r, wroot3r = row(Wroot1), row(Wroot2), row(Wroot3)
    wv2r = row(Wv2)
    brel1r, brel2r, brel3r = row(brel1), row(brel2), row(brel3)
    bv2r = row(bv2)
    wlt, wlb = Wl[:512], Wl[512:]

    a, r1, hsum, asum = pl.pallas_call(
        _k1_body,
        grid=(GRID,),
        in_specs=[pl.BlockSpec((TILE, 256), lambda t: (t, 0)),
                  _FULL((256, 512)), _FULL((1, 512)), _FULL((1, 512))],
        out_specs=[_ROWS, _SCORES, _FULL((1, 512)), _FULL((1, 512))],
        out_shape=[jax.ShapeDtypeStruct((NP, 512), f32),
                   jax.ShapeDtypeStruct((NP, 1), f32),
                   _vec(512), _vec(512)],
    )(obs_p, W1, b1r, wroot1r)

    t1, c1, sig1, yc1, h2c = pl.pallas_call(
        functools.partial(_c1_body, K1),
        in_specs=[pl.BlockSpec((NP, 1), lambda: (0, 0))]
        + [pl.BlockSpec(s, lambda: (0, 0)) for s in
           [(1, 512), (1, 512), (1, 512), (1, 512), (1, 1), (1, 512), (512, 512)]][:7],
        out_specs=[pl.BlockSpec((1, 1), lambda: (0, 0))] * 3
        + [pl.BlockSpec((1, 512), lambda: (0, 0))] * 2,
        out_shape=[_SCAL, _SCAL, _SCAL, _vec(512), _vec(512)],
    )(r1, hsum, asum, b1r, wrel1r, brel1r, wroot1r, W2)

    x2, r2, h2sum, x2sum, mx1, sm1 = pl.pallas_call(
        functools.partial(_mid_body, K1, float(K1 + 1)),
        grid=(GRID,),
        in_specs=[_ROWS, _SCORES, _FULL((512, 512)), _FULL((1, 512)),
                  _FULL((1, 512)), _FULL((1, 1)), _FULL((1, 1)), _FULL((1, 1)),
                  _FULL((1, 512))],
        out_specs=[_ROWS, _SCORES] + [_FULL((1, 512))] * 4,
        out_shape=[jax.ShapeDtypeStruct((NP, 512), f32),
                   jax.ShapeDtypeStruct((NP, 1), f32),
                   _vec(512), _vec(512), _vec(512), _vec(512)],
    )(a, r1, W2, b2r, wroot2r, sig1, t1, c1, h2c)

    t2, c2, sig2, yc2, h3c = pl.pallas_call(
        functools.partial(_cn_body, K2, float(K1 + 1)),
        in_specs=[pl.BlockSpec((NP, 1), lambda: (0, 0))]
        + [pl.BlockSpec(s, lambda: (0, 0)) for s in
           [(1, 512), (1, 512), (1, 512), (1, 1), (1, 512), (1, 512), (1, 1),
            (1, 512), (512, 512)]],
        out_specs=[pl.BlockSpec((1, 1), lambda: (0, 0))] * 3
        + [pl.BlockSpec((1, 512), lambda: (0, 0))] * 2,
        out_shape=[_SCAL, _SCAL, _SCAL, _vec(512), _vec(512)],
    )(r2, h2sum, x2sum, h2c, c1, b2r, wrel2r, brel2r, wroot2r, W3)

    x3, r3, h3sum, x3sum, mx2, sm2 = pl.pallas_call(
        functools.partial(_mid_body, K2, float(K2 + 1)),
        grid=(GRID,),
        in_specs=[_ROWS, _SCORES, _FULL((512, 512)), _FULL((1, 512)),
                  _FULL((1, 512)), _FULL((1, 1)), _FULL((1, 1)), _FULL((1, 1)),
                  _FULL((1, 512))],
        out_specs=[_ROWS, _SCORES] + [_FULL((1, 512))] * 4,
        out_shape=[jax.ShapeDtypeStruct((NP, 512), f32),
                   jax.ShapeDtypeStruct((NP, 1), f32),
                   _vec(512), _vec(512), _vec(512), _vec(512)],
    )(x2, r2, W3, b3r, wroot3r, sig2, t2, c2, h3c)

    t3, c3, sig3, yc3, _ = pl.pallas_call(
        functools.partial(_cn_body, K3, float(K2 + 1)),
        in_specs=[pl.BlockSpec((NP, 1), lambda: (0, 0))]
        + [pl.BlockSpec(s, lambda: (0, 0)) for s in
           [(1, 512), (1, 512), (1, 512), (1, 1), (1, 512), (1, 512), (1, 1),
            (1, 512), (512, 512)]],
        out_specs=[pl.BlockSpec((1, 1), lambda: (0, 0))] * 3
        + [pl.BlockSpec((1, 512), lambda: (0, 0))] * 2,
        out_shape=[_SCAL, _SCAL, _SCAL, _vec(512), _vec(512)],
    )(r3, h3sum, x3sum, h3c, c2, b3r, wrel3r, brel3r, wroot3r, W3)

    mx3, sm3 = pl.pallas_call(
        _k7_body,
        grid=(GRID,),
        in_specs=[_ROWS, _SCORES, _FULL((1, 1)), _FULL((1, 1))],
        out_specs=[_FULL((1, 512))] * 2,
        out_shape=[_vec(512), _vec(512)],
    )(x3, r3, sig3, t3)

    v = pl.pallas_call(
        _k8_body,
        in_specs=[pl.BlockSpec(s, lambda: (0, 0)) for s in
                  [(1, 512)] * 6 + [(1, 512)] * 3 + [(1, 1)] * 3
                  + [(512, 512), (512, 512), (1, 512), (512, 512), (1, 512),
                     (1, 512), (1, 1)]],
        out_specs=pl.BlockSpec((1, 1), lambda: (0, 0)),
        out_shape=_SCAL,
    )(mx1, sm1, mx2, sm2, mx3, sm3, yc1, yc2, yc3, c1, c2, c3,
      wlt, wlb, blr, Wv1, bv1r, wv2r, bv2r)

    return v.reshape(1)


# speedup vs baseline: 46.9430x; 2.2990x over previous
import jax
import jax.numpy as jnp
import numpy as np
from jax import lax
from jax.experimental import pallas as pl
from jax.experimental.pallas import tpu as pltpu

N = 10000
NP = 10240
TILE = 1024
GRID = NP // TILE
K1, K2, K3 = 5001, 2501, 1251
NEG = float(np.float32(-1e30))

_DISA1 = np.float32(2.0) ** np.float32(-0.5)
_DIS01 = np.float32(N + 1.0) ** np.float32(-0.5)
SELF1 = float(np.float32(_DISA1 * _DISA1))
CROSS1 = float(np.float32(_DISA1 * _DIS01))

f32 = jnp.float32
bf16 = jnp.bfloat16


def _kth_threshold(sarr, s_extra, k):
    b = lax.bitcast_convert_type(sarr, jnp.uint32)
    u = b ^ jnp.where(b >> 31 == 1, jnp.uint32(0xFFFFFFFF), jnp.uint32(0x80000000))
    bx = lax.bitcast_convert_type(s_extra, jnp.uint32)
    ux = bx ^ jnp.where(bx >> 31 == 1, jnp.uint32(0xFFFFFFFF), jnp.uint32(0x80000000))

    def body(i, t):
        cand = t | (jnp.uint32(1) << (jnp.uint32(31) - jnp.uint32(i)))
        cnt = jnp.sum((u >= cand).astype(jnp.int32)) + (ux >= cand).astype(jnp.int32)
        return jnp.where(cnt >= k, cand, t)

    t = lax.fori_loop(0, 32, body, jnp.uint32(0))
    tb = jnp.where(t & jnp.uint32(0x80000000) != 0,
                   t ^ jnp.uint32(0x80000000), ~t)
    return lax.bitcast_convert_type(tb, jnp.float32)


def _mega_body(obs_ref, w1_ref, b1_ref, wq1_ref, brel1_ref,
               w2_ref, b2_ref, wq2_ref, brel2_ref,
               w3_ref, b3_ref, wq3_ref, brel3_ref,
               wlt_ref, wlb_ref, bl_ref, wv1_ref, bv1_ref, wv2_ref, bv2_ref,
               v_ref,
               A, B, rr1, rr2, rr3, hs, fs,
               sm1, mx1, sm2, mx2, sm3, mx3,
               sig, thr, c1r, c2r, c3r, yc1, yc2, yc3, hc):
    t = pl.program_id(0)

    def sag_epilogue(xc, rr_ref, wq_ref, brel_s, wnext_ref, cprev, kk,
                     cdst_ref, ycdst_ref):
        wroot = wq_ref[0:1, :].astype(f32)
        wrel = wq_ref[1:2, :].astype(f32)
        xcb = xc.astype(bf16).astype(f32)
        fsb = fs[...].astype(bf16).astype(f32)
        sg = cprev * jnp.sum(xcb * wrel) + brel_s
        sc_live = (jnp.sum(fsb * wrel) + brel_s + jnp.sum(xcb * wroot))
        sc = jnp.where(cprev > 0, sc_live, NEG)
        th = _kth_threshold(sg + rr_ref[...], sc, kk)
        c = (sc >= th).astype(f32)
        yc = xc * (jnp.tanh(sc) * c)
        sig[...] = jnp.reshape(sg, (1, 1))
        thr[...] = jnp.reshape(th, (1, 1))
        cdst_ref[...] = jnp.reshape(c, (1, 1))
        ycdst_ref[...] = yc
        hc[...] = jnp.dot(yc.astype(bf16), wnext_ref[...],
                          preferred_element_type=f32)

    @pl.when(t < GRID)
    def _():
        tile = t
        rows = pl.ds(pl.multiple_of(tile * TILE, TILE), TILE)
        obs = obs_ref[...]
        h = jnp.dot(obs, w1_ref[...], preferred_element_type=f32)
        rid = tile * TILE + lax.broadcasted_iota(jnp.int32, (TILE, 1), 0)
        valid = rid < N
        a = jnp.maximum(h * SELF1 + b1_ref[...], 0.0)
        a = jnp.where(valid, a, 0.0)
        A[rows, :] = a
        proj = lax.dot_general(a.astype(bf16), wq1_ref[...],
                               (((1,), (1,)), ((), ())),
                               preferred_element_type=f32)
        rm = jnp.where(valid, proj[:, 0:1], NEG)
        rr1[:, rows] = rm.T

        @pl.when(tile == 0)
        def _():
            hs[...] = jnp.zeros_like(hs)
            fs[...] = jnp.zeros_like(fs)

        hs[...] += jnp.sum(h, axis=0, keepdims=True)
        fs[...] += jnp.sum(a, axis=0, keepdims=True)

        @pl.when(tile == GRID - 1)
        def _():
            xc = jnp.maximum(hs[...] * CROSS1 + b1_ref[...], 0.0)
            sag_epilogue(xc, rr1, wq1_ref, brel1_ref[0, 0], w2_ref,
                         jnp.float32(1.0), K1, c1r, yc1)

    def mid_phase(tile, src, dst, rrg, rrn, w_ref, b_ref, wq_ref, brel_ref,
                  wnext_ref, cp_ref, sm, mx, degc_base, kk, cdst_ref,
                  ycdst_ref, last):
        rows = pl.ds(pl.multiple_of(tile * TILE, TILE), TILE)
        aa = src[rows, :]
        c = cp_ref[0, 0]
        s_row = sig[0, 0] + rrg[:, rows]
        alive_row = s_row >= thr[0, 0]
        g_row = jnp.where(alive_row, jnp.tanh(s_row), 0.0)
        am_row = jnp.where(alive_row, 1.0, 0.0)
        g = g_row.T
        am = am_row.T
        y = aa * g
        h = jnp.dot(y.astype(bf16), w_ref[...], preferred_element_type=f32)
        disc = lax.rsqrt(jnp.float32(degc_base) - c)
        disa = lax.rsqrt(1.0 + c)
        out = (c * (disc * disa)) * hc[...] + h * (disa * disa) + b_ref[...]
        x = jnp.maximum(out, 0.0) * am
        dst[rows, :] = x
        proj = lax.dot_general(x.astype(bf16), wq_ref[...],
                               (((1,), (1,)), ((), ())),
                               preferred_element_type=f32)
        rm = jnp.where(am > 0, proj[:, 0:1], NEG)
        rrn[:, rows] = rm.T

        @pl.when(tile == 0)
        def _():
            sm[...] = jnp.zeros_like(sm)
            mx[...] = jnp.full_like(mx, NEG)
            hs[...] = jnp.zeros_like(hs)
            fs[...] = jnp.zeros_like(fs)

        sm[...] += jnp.sum(y, axis=0, keepdims=True)
        mx[...] = jnp.maximum(mx[...],
                              jnp.max(jnp.where(am > 0, y, NEG), axis=0,
                                      keepdims=True))
        hs[...] += jnp.sum(h, axis=0, keepdims=True)
        fs[...] += jnp.sum(x, axis=0, keepdims=True)

        @pl.when(tile == GRID - 1)
        def _():
            disc_e = lax.rsqrt(jnp.float32(degc_base) - c)
            disa_e = lax.rsqrt(1.0 + c)
            xc = jnp.maximum((disc_e * disa_e) * hs[...]
                             + (disc_e * disc_e) * hc[...] + b_ref[...],
                             0.0) * c
            sag_epilogue(xc, rrn, wq_ref, brel_ref[0, 0], wnext_ref, c, kk,
                         cdst_ref, ycdst_ref)

    @pl.when((t >= GRID) & (t < 2 * GRID))
    def _():
        mid_phase(t - GRID, A, B, rr1, rr2, w2_ref, b2_ref, wq2_ref,
                  brel2_ref, w3_ref, c1r, sm1, mx1, float(K1 + 1), K2,
                  c2r, yc2, False)

    @pl.when((t >= 2 * GRID) & (t < 3 * GRID))
    def _():
        mid_phase(t - 2 * GRID, B, A, rr2, rr3, w3_ref, b3_ref, wq3_ref,
                  brel3_ref, w3_ref, c2r, sm2, mx2, float(K2 + 1), K3,
                  c3r, yc3, False)

    @pl.when(t >= 3 * GRID)
    def _():
        tile = t - 3 * GRID
        rows = pl.ds(pl.multiple_of(tile * TILE, TILE), TILE)
        xx = A[rows, :]
        s_row = sig[0, 0] + rr3[:, rows]
        alive_row = s_row >= thr[0, 0]
        g_row = jnp.where(alive_row, jnp.tanh(s_row), 0.0)
        am_row = jnp.where(alive_row, 1.0, 0.0)
        y = xx * g_row.T
        am = am_row.T

        @pl.when(tile == 0)
        def _():
            sm3[...] = jnp.zeros_like(sm3)
            mx3[...] = jnp.full_like(mx3, NEG)

        sm3[...] += jnp.sum(y, axis=0, keepdims=True)
        mx3[...] = jnp.maximum(mx3[...],
                               jnp.max(jnp.where(am > 0, y, NEG), axis=0,
                                       keepdims=True))

        @pl.when(tile == GRID - 1)
        def _():
            def merge(mx, sm, yc_ref, c_ref, k):
                cc = c_ref[0, 0]
                m = jnp.maximum(mx, jnp.where(cc > 0, yc_ref[...], NEG))
                return m, (sm + yc_ref[...]) / jnp.float32(k)

            m1, a1 = merge(mx1[...], sm1[...], yc1, c1r, K1)
            m2, a2 = merge(mx2[...], sm2[...], yc2, c2r, K2)
            m3, a3 = merge(mx3[...], sm3[...], yc3, c3r, K3)
            state = (jnp.dot((m1 + m2 + m3).astype(bf16), wlt_ref[...],
                             preferred_element_type=f32)
                     + jnp.dot((a1 + a2 + a3).astype(bf16), wlb_ref[...],
                               preferred_element_type=f32)
                     + bl_ref[...])
            sv = jnp.maximum(jnp.dot(state.astype(bf16), wv1_ref[...],
                                     preferred_element_type=f32)
                             + bv1_ref[...], 0.0)
            svb = sv.astype(bf16).astype(f32)
            v_ref[...] = jnp.reshape(jnp.sum(svb * wv2_ref[...])
                                     + bv2_ref[0, 0], (1, 1))


def kernel(obs, is_alive, W1, b1, Wrel1, brel1, Wroot1, W2, b2, Wrel2, brel2,
           Wroot2, W3, b3, Wrel3, brel3, Wroot3, Wl, bl, Wv1, bv1, Wv2, bv2):
    obs_p = jnp.pad(obs, ((0, NP - N), (0, 0))).astype(bf16)
    row = lambda w: w.reshape(1, -1).astype(f32)
    b1r, b2r, b3r, blr, bv1r = row(b1), row(b2), row(b3), row(bl), row(bv1)
    brel1r, brel2r, brel3r, bv2r = (row(brel1), row(brel2), row(brel3),
                                    row(bv2))
    wv2r = row(Wv2).astype(bf16).astype(f32)
    wq1 = jnp.concatenate([row(Wroot1), row(Wrel1)], axis=0).astype(bf16)
    wq2 = jnp.concatenate([row(Wroot2), row(Wrel2)], axis=0).astype(bf16)
    wq3 = jnp.concatenate([row(Wroot3), row(Wrel3)], axis=0).astype(bf16)
    w1b, w2b, w3b = W1.astype(bf16), W2.astype(bf16), W3.astype(bf16)
    wv1b = Wv1.astype(bf16)
    wlt, wlb = Wl[:512].astype(bf16), Wl[512:].astype(bf16)

    _C = lambda: pl.BlockSpec((1, 1), lambda t: (0, 0))
    _V = lambda n: pl.BlockSpec((1, n), lambda t: (0, 0))
    _F = lambda s: pl.BlockSpec(s, lambda t: tuple(0 for _ in s))

    v = pl.pallas_call(
        _mega_body,
        grid=(4 * GRID,),
        in_specs=[pl.BlockSpec((TILE, 256), lambda t: (jnp.where(t < GRID, t, 0), 0)),
                  _F((256, 512)), _V(512), _F((2, 512)), _C(),
                  _F((512, 512)), _V(512), _F((2, 512)), _C(),
                  _F((512, 512)), _V(512), _F((2, 512)), _C(),
                  _F((512, 512)), _F((512, 512)), _V(512),
                  _F((512, 512)), _V(512), _V(512), _C()],
        out_specs=_C(),
        out_shape=jax.ShapeDtypeStruct((1, 1), f32),
        scratch_shapes=[
            pltpu.VMEM((NP, 512), f32), pltpu.VMEM((NP, 512), f32),
            pltpu.VMEM((1, NP), f32), pltpu.VMEM((1, NP), f32),
            pltpu.VMEM((1, NP), f32),
            pltpu.VMEM((1, 512), f32), pltpu.VMEM((1, 512), f32),
            pltpu.VMEM((1, 512), f32), pltpu.VMEM((1, 512), f32),
            pltpu.VMEM((1, 512), f32), pltpu.VMEM((1, 512), f32),
            pltpu.VMEM((1, 512), f32), pltpu.VMEM((1, 512), f32),
            pltpu.VMEM((1, 1), f32), pltpu.VMEM((1, 1), f32),
            pltpu.VMEM((1, 1), f32), pltpu.VMEM((1, 1), f32),
            pltpu.VMEM((1, 1), f32),
            pltpu.VMEM((1, 512), f32), pltpu.VMEM((1, 512), f32),
            pltpu.VMEM((1, 512), f32), pltpu.VMEM((1, 512), f32),
        ],
        compiler_params=pltpu.CompilerParams(
            dimension_semantics=("arbitrary",)),
    )(obs_p, w1b, b1r, wq1, brel1r, w2b, b2r, wq2, brel2r, w3b, b3r, wq3,
      brel3r, wlt, wlb, blr, wv1b, bv1r, wv2r, bv2r)

    return v.reshape(1)


# speedup vs baseline: 53.0596x; 1.1303x over previous
import jax
import jax.numpy as jnp
import numpy as np
from jax import lax
from jax.experimental import pallas as pl
from jax.experimental.pallas import tpu as pltpu

N = 10000
NP = 10240
TILE = 2048
GRID = NP // TILE
K1, K2, K3 = 5001, 2501, 1251
NEG = float(np.float32(-1e30))

_DISA1 = np.float32(2.0) ** np.float32(-0.5)
_DIS01 = np.float32(N + 1.0) ** np.float32(-0.5)
SELF1 = float(np.float32(_DISA1 * _DISA1))
CROSS1 = float(np.float32(_DISA1 * _DIS01))

f32 = jnp.float32
bf16 = jnp.bfloat16


def _kth_threshold(sarr, s_extra, k):
    b = lax.bitcast_convert_type(sarr, jnp.uint32)
    u = b ^ jnp.where(b >> 31 == 1, jnp.uint32(0xFFFFFFFF), jnp.uint32(0x80000000))
    bx = lax.bitcast_convert_type(s_extra, jnp.uint32)
    ux = bx ^ jnp.where(bx >> 31 == 1, jnp.uint32(0xFFFFFFFF), jnp.uint32(0x80000000))

    def body(i, t):
        cand = t | (jnp.uint32(1) << (jnp.uint32(31) - jnp.uint32(i)))
        cnt = jnp.sum((u >= cand).astype(jnp.int32)) + (ux >= cand).astype(jnp.int32)
        return jnp.where(cnt >= k, cand, t)

    t = lax.fori_loop(0, 32, body, jnp.uint32(0))
    tb = jnp.where(t & jnp.uint32(0x80000000) != 0,
                   t ^ jnp.uint32(0x80000000), ~t)
    return lax.bitcast_convert_type(tb, jnp.float32)


def _mega_body(obs_ref, w1_ref, b1_ref, wq1_ref, brel1_ref,
               w2_ref, b2_ref, wq2_ref, brel2_ref,
               w3_ref, b3_ref, wq3_ref, brel3_ref,
               wlt_ref, wlb_ref, bl_ref, wv1_ref, bv1_ref, wv2_ref, bv2_ref,
               v_ref,
               A, rr1, rr2, rr3, hs, fs,
               sm1, mx1, sm2, mx2, sm3, mx3,
               sig, thr, c1r, c2r, c3r, yc1, yc2, yc3, hc):
    t = pl.program_id(0)

    def sag_epilogue(xc, rr_ref, wq_ref, brel_s, wnext_ref, cprev, kk,
                     cdst_ref, ycdst_ref):
        wroot = wq_ref[0:1, :].astype(f32)
        wrel = wq_ref[1:2, :].astype(f32)
        xcb = xc.astype(bf16).astype(f32)
        fsb = fs[...].astype(bf16).astype(f32)
        sg = cprev * jnp.sum(xcb * wrel) + brel_s
        sc_live = (jnp.sum(fsb * wrel) + brel_s + jnp.sum(xcb * wroot))
        sc = jnp.where(cprev > 0, sc_live, NEG)
        th = _kth_threshold(sg + rr_ref[...], sc, kk)
        c = (sc >= th).astype(f32)
        yc = xc * (jnp.tanh(sc) * c)
        sig[...] = jnp.reshape(sg, (1, 1))
        thr[...] = jnp.reshape(th, (1, 1))
        cdst_ref[...] = jnp.reshape(c, (1, 1))
        ycdst_ref[...] = yc
        hc[...] = jnp.dot(yc.astype(bf16), wnext_ref[...],
                          preferred_element_type=f32)

    @pl.when(t < GRID)
    def _():
        tile = t
        rows = pl.ds(pl.multiple_of(tile * TILE, TILE), TILE)
        obs = obs_ref[...]
        h = jnp.dot(obs, w1_ref[...], preferred_element_type=f32)
        rid = tile * TILE + lax.broadcasted_iota(jnp.int32, (TILE, 1), 0)
        valid = rid < N
        a = jnp.maximum(h * SELF1 + b1_ref[...], 0.0)
        a = jnp.where(valid, a, 0.0)
        A[rows, :] = a
        proj = lax.dot_general(a.astype(bf16), wq1_ref[...],
                               (((1,), (1,)), ((), ())),
                               preferred_element_type=f32)
        rm = jnp.where(valid, proj[:, 0:1], NEG)
        rr1[:, rows] = rm.T

        @pl.when(tile == 0)
        def _():
            hs[...] = jnp.zeros_like(hs)
            fs[...] = jnp.zeros_like(fs)

        hs[...] += jnp.sum(h, axis=0, keepdims=True)
        fs[...] += jnp.sum(a, axis=0, keepdims=True)

        @pl.when(tile == GRID - 1)
        def _():
            xc = jnp.maximum(hs[...] * CROSS1 + b1_ref[...], 0.0)
            sag_epilogue(xc, rr1, wq1_ref, brel1_ref[0, 0], w2_ref,
                         jnp.float32(1.0), K1, c1r, yc1)

    def mid_phase(tile, src, dst, rrg, rrn, w_ref, b_ref, wq_ref, brel_ref,
                  wnext_ref, cp_ref, sm, mx, degc_base, kk, cdst_ref,
                  ycdst_ref, last):
        rows = pl.ds(pl.multiple_of(tile * TILE, TILE), TILE)
        aa = src[rows, :]
        c = cp_ref[0, 0]
        s_row = sig[0, 0] + rrg[:, rows]
        alive_row = s_row >= thr[0, 0]
        g_row = jnp.where(alive_row, jnp.tanh(s_row), 0.0)
        am_row = jnp.where(alive_row, 1.0, 0.0)
        g = g_row.T
        am = am_row.T
        y = aa * g
        h = jnp.dot(y.astype(bf16), w_ref[...], preferred_element_type=f32)
        disc = lax.rsqrt(jnp.float32(degc_base) - c)
        disa = lax.rsqrt(1.0 + c)
        out = (c * (disc * disa)) * hc[...] + h * (disa * disa) + b_ref[...]
        x = jnp.maximum(out, 0.0) * am
        dst[rows, :] = x
        proj = lax.dot_general(x.astype(bf16), wq_ref[...],
                               (((1,), (1,)), ((), ())),
                               preferred_element_type=f32)
        rm = jnp.where(am > 0, proj[:, 0:1], NEG)
        rrn[:, rows] = rm.T

        @pl.when(tile == 0)
        def _():
            sm[...] = jnp.zeros_like(sm)
            mx[...] = jnp.full_like(mx, NEG)
            hs[...] = jnp.zeros_like(hs)
            fs[...] = jnp.zeros_like(fs)

        sm[...] += jnp.sum(y, axis=0, keepdims=True)
        mx[...] = jnp.maximum(mx[...],
                              jnp.max(jnp.where(am > 0, y, NEG), axis=0,
                                      keepdims=True))
        hs[...] += jnp.sum(h, axis=0, keepdims=True)
        fs[...] += jnp.sum(x, axis=0, keepdims=True)

        @pl.when(tile == GRID - 1)
        def _():
            disc_e = lax.rsqrt(jnp.float32(degc_base) - c)
            disa_e = lax.rsqrt(1.0 + c)
            xc = jnp.maximum((disc_e * disa_e) * hs[...]
                             + (disc_e * disc_e) * hc[...] + b_ref[...],
                             0.0) * c
            sag_epilogue(xc, rrn, wq_ref, brel_ref[0, 0], wnext_ref, c, kk,
                         cdst_ref, ycdst_ref)

    @pl.when((t >= GRID) & (t < 2 * GRID))
    def _():
        mid_phase(t - GRID, A, A, rr1, rr2, w2_ref, b2_ref, wq2_ref,
                  brel2_ref, w3_ref, c1r, sm1, mx1, float(K1 + 1), K2,
                  c2r, yc2, False)

    @pl.when((t >= 2 * GRID) & (t < 3 * GRID))
    def _():
        mid_phase(t - 2 * GRID, A, A, rr2, rr3, w3_ref, b3_ref, wq3_ref,
                  brel3_ref, w3_ref, c2r, sm2, mx2, float(K2 + 1), K3,
                  c3r, yc3, False)

    @pl.when(t >= 3 * GRID)
    def _():
        tile = t - 3 * GRID
        rows = pl.ds(pl.multiple_of(tile * TILE, TILE), TILE)
        xx = A[rows, :]
        s_row = sig[0, 0] + rr3[:, rows]
        alive_row = s_row >= thr[0, 0]
        g_row = jnp.where(alive_row, jnp.tanh(s_row), 0.0)
        am_row = jnp.where(alive_row, 1.0, 0.0)
        y = xx * g_row.T
        am = am_row.T

        @pl.when(tile == 0)
        def _():
            sm3[...] = jnp.zeros_like(sm3)
            mx3[...] = jnp.full_like(mx3, NEG)

        sm3[...] += jnp.sum(y, axis=0, keepdims=True)
        mx3[...] = jnp.maximum(mx3[...],
                               jnp.max(jnp.where(am > 0, y, NEG), axis=0,
                                       keepdims=True))

        @pl.when(tile == GRID - 1)
        def _():
            def merge(mx, sm, yc_ref, c_ref, k):
                cc = c_ref[0, 0]
                m = jnp.maximum(mx, jnp.where(cc > 0, yc_ref[...], NEG))
                return m, (sm + yc_ref[...]) / jnp.float32(k)

            m1, a1 = merge(mx1[...], sm1[...], yc1, c1r, K1)
            m2, a2 = merge(mx2[...], sm2[...], yc2, c2r, K2)
            m3, a3 = merge(mx3[...], sm3[...], yc3, c3r, K3)
            state = (jnp.dot((m1 + m2 + m3).astype(bf16), wlt_ref[...],
                             preferred_element_type=f32)
                     + jnp.dot((a1 + a2 + a3).astype(bf16), wlb_ref[...],
                               preferred_element_type=f32)
                     + bl_ref[...])
            sv = jnp.maximum(jnp.dot(state.astype(bf16), wv1_ref[...],
                                     preferred_element_type=f32)
                             + bv1_ref[...], 0.0)
            svb = sv.astype(bf16).astype(f32)
            v_ref[...] = jnp.reshape(jnp.sum(svb * wv2_ref[...])
                                     + bv2_ref[0, 0], (1, 1))


def kernel(obs, is_alive, W1, b1, Wrel1, brel1, Wroot1, W2, b2, Wrel2, brel2,
           Wroot2, W3, b3, Wrel3, brel3, Wroot3, Wl, bl, Wv1, bv1, Wv2, bv2):
    obs_p = jnp.pad(obs, ((0, NP - N), (0, 0))).astype(bf16)
    row = lambda w: w.reshape(1, -1).astype(f32)
    b1r, b2r, b3r, blr, bv1r = row(b1), row(b2), row(b3), row(bl), row(bv1)
    brel1r, brel2r, brel3r, bv2r = (row(brel1), row(brel2), row(brel3),
                                    row(bv2))
    wv2r = row(Wv2).astype(bf16).astype(f32)
    wq1 = jnp.concatenate([row(Wroot1), row(Wrel1)], axis=0).astype(bf16)
    wq2 = jnp.concatenate([row(Wroot2), row(Wrel2)], axis=0).astype(bf16)
    wq3 = jnp.concatenate([row(Wroot3), row(Wrel3)], axis=0).astype(bf16)
    w1b, w2b, w3b = W1.astype(bf16), W2.astype(bf16), W3.astype(bf16)
    wv1b = Wv1.astype(bf16)
    wlt, wlb = Wl[:512].astype(bf16), Wl[512:].astype(bf16)

    _C = lambda: pl.BlockSpec((1, 1), lambda t: (0, 0))
    _V = lambda n: pl.BlockSpec((1, n), lambda t: (0, 0))
    _F = lambda s: pl.BlockSpec(s, lambda t: tuple(0 for _ in s))

    v = pl.pallas_call(
        _mega_body,
        grid=(4 * GRID,),
        in_specs=[pl.BlockSpec((TILE, 256), lambda t: (jnp.where(t < GRID, t, 0), 0)),
                  _F((256, 512)), _V(512), _F((2, 512)), _C(),
                  _F((512, 512)), _V(512), _F((2, 512)), _C(),
                  _F((512, 512)), _V(512), _F((2, 512)), _C(),
                  _F((512, 512)), _F((512, 512)), _V(512),
                  _F((512, 512)), _V(512), _V(512), _C()],
        out_specs=_C(),
        out_shape=jax.ShapeDtypeStruct((1, 1), f32),
        scratch_shapes=[
            pltpu.VMEM((NP, 512), f32),
            pltpu.VMEM((1, NP), f32), pltpu.VMEM((1, NP), f32),
            pltpu.VMEM((1, NP), f32),
            pltpu.VMEM((1, 512), f32), pltpu.VMEM((1, 512), f32),
            pltpu.VMEM((1, 512), f32), pltpu.VMEM((1, 512), f32),
            pltpu.VMEM((1, 512), f32), pltpu.VMEM((1, 512), f32),
            pltpu.VMEM((1, 512), f32), pltpu.VMEM((1, 512), f32),
            pltpu.VMEM((1, 1), f32), pltpu.VMEM((1, 1), f32),
            pltpu.VMEM((1, 1), f32), pltpu.VMEM((1, 1), f32),
            pltpu.VMEM((1, 1), f32),
            pltpu.VMEM((1, 512), f32), pltpu.VMEM((1, 512), f32),
            pltpu.VMEM((1, 512), f32), pltpu.VMEM((1, 512), f32),
        ],
        compiler_params=pltpu.CompilerParams(
            dimension_semantics=("arbitrary",)),
    )(obs_p, w1b, b1r, wq1, brel1r, w2b, b2r, wq2, brel2r, w3b, b3r, wq3,
      brel3r, wlt, wlb, blr, wv1b, bv1r, wv2r, bv2r)

    return v.reshape(1)


# speedup vs baseline: 54.7944x; 1.0327x over previous
import jax
import jax.numpy as jnp
import numpy as np
from jax import lax
from jax.experimental import pallas as pl
from jax.experimental.pallas import tpu as pltpu

N = 10000
NP = 10240
TILE = 2560
GRID = NP // TILE
K1, K2, K3 = 5001, 2501, 1251
NEG = float(np.float32(-1e30))

_DISA1 = np.float32(2.0) ** np.float32(-0.5)
_DIS01 = np.float32(N + 1.0) ** np.float32(-0.5)
SELF1 = float(np.float32(_DISA1 * _DISA1))
CROSS1 = float(np.float32(_DISA1 * _DIS01))

f32 = jnp.float32
bf16 = jnp.bfloat16


def _kth_threshold(sarr, s_extra, k):
    b = lax.bitcast_convert_type(sarr, jnp.uint32)
    u = b ^ jnp.where(b >> 31 == 1, jnp.uint32(0xFFFFFFFF), jnp.uint32(0x80000000))
    bx = lax.bitcast_convert_type(s_extra, jnp.uint32)
    ux = bx ^ jnp.where(bx >> 31 == 1, jnp.uint32(0xFFFFFFFF), jnp.uint32(0x80000000))

    def body(i, t):
        cand = t | (jnp.uint32(1) << (jnp.uint32(31) - jnp.uint32(i)))
        cnt = jnp.sum((u >= cand).astype(jnp.int32)) + (ux >= cand).astype(jnp.int32)
        return jnp.where(cnt >= k, cand, t)

    t = lax.fori_loop(0, 32, body, jnp.uint32(0))
    tb = jnp.where(t & jnp.uint32(0x80000000) != 0,
                   t ^ jnp.uint32(0x80000000), ~t)
    return lax.bitcast_convert_type(tb, jnp.float32)


def _mega_body(obs_ref, w1_ref, b1_ref, wq1_ref, brel1_ref,
               w2_ref, b2_ref, wq2_ref, brel2_ref,
               w3_ref, b3_ref, wq3_ref, brel3_ref,
               wlt_ref, wlb_ref, bl_ref, wv1_ref, bv1_ref, wv2_ref, bv2_ref,
               v_ref,
               A, rr1, rr2, rr3, hs, fs,
               sm1, mx1, sm2, mx2, sm3, mx3,
               sig, thr, c1r, c2r, c3r, yc1, yc2, yc3, hc):
    t = pl.program_id(0)

    def sag_epilogue(xc, rr_ref, wq_ref, brel_s, wnext_ref, cprev, kk,
                     cdst_ref, ycdst_ref):
        wroot = wq_ref[0:1, :].astype(f32)
        wrel = wq_ref[1:2, :].astype(f32)
        xcb = xc.astype(bf16).astype(f32)
        fsb = fs[...].astype(bf16).astype(f32)
        sg = cprev * jnp.sum(xcb * wrel) + brel_s
        sc_live = (jnp.sum(fsb * wrel) + brel_s + jnp.sum(xcb * wroot))
        sc = jnp.where(cprev > 0, sc_live, NEG)
        th = _kth_threshold(sg + rr_ref[...], sc, kk)
        c = (sc >= th).astype(f32)
        yc = xc * (jnp.tanh(sc) * c)
        sig[...] = jnp.reshape(sg, (1, 1))
        thr[...] = jnp.reshape(th, (1, 1))
        cdst_ref[...] = jnp.reshape(c, (1, 1))
        ycdst_ref[...] = yc
        hc[...] = jnp.dot(yc.astype(bf16), wnext_ref[...],
                          preferred_element_type=f32)

    @pl.when(t < GRID)
    def _():
        tile = t
        rows = pl.ds(pl.multiple_of(tile * TILE, TILE), TILE)
        obs = obs_ref[...]
        h = jnp.dot(obs, w1_ref[...], preferred_element_type=f32)
        rid = tile * TILE + lax.broadcasted_iota(jnp.int32, (TILE, 1), 0)
        valid = rid < N
        a = jnp.maximum(h * SELF1 + b1_ref[...], 0.0)
        a = jnp.where(valid, a, 0.0)
        A[rows, :] = a
        proj = lax.dot_general(a.astype(bf16), wq1_ref[...],
                               (((1,), (1,)), ((), ())),
                               preferred_element_type=f32)
        rm = jnp.where(valid, proj[:, 0:1], NEG)
        rr1[:, rows] = rm.T

        @pl.when(tile == 0)
        def _():
            hs[...] = jnp.zeros_like(hs)
            fs[...] = jnp.zeros_like(fs)

        hs[...] += jnp.sum(h, axis=0, keepdims=True)
        fs[...] += jnp.sum(a, axis=0, keepdims=True)

        @pl.when(tile == GRID - 1)
        def _():
            xc = jnp.maximum(hs[...] * CROSS1 + b1_ref[...], 0.0)
            sag_epilogue(xc, rr1, wq1_ref, brel1_ref[0, 0], w2_ref,
                         jnp.float32(1.0), K1, c1r, yc1)

    def mid_phase(tile, src, dst, rrg, rrn, w_ref, b_ref, wq_ref, brel_ref,
                  wnext_ref, cp_ref, sm, mx, degc_base, kk, cdst_ref,
                  ycdst_ref, last):
        rows = pl.ds(pl.multiple_of(tile * TILE, TILE), TILE)
        aa = src[rows, :]
        c = cp_ref[0, 0]
        s_row = sig[0, 0] + rrg[:, rows]
        alive_row = s_row >= thr[0, 0]
        g_row = jnp.where(alive_row, jnp.tanh(s_row), 0.0)
        am_row = jnp.where(alive_row, 1.0, 0.0)
        g = g_row.T
        am = am_row.T
        y = aa * g
        h = jnp.dot(y.astype(bf16), w_ref[...], preferred_element_type=f32)
        disc = lax.rsqrt(jnp.float32(degc_base) - c)
        disa = lax.rsqrt(1.0 + c)
        out = (c * (disc * disa)) * hc[...] + h * (disa * disa) + b_ref[...]
        x = jnp.maximum(out, 0.0) * am
        dst[rows, :] = x
        proj = lax.dot_general(x.astype(bf16), wq_ref[...],
                               (((1,), (1,)), ((), ())),
                               preferred_element_type=f32)
        rm = jnp.where(am > 0, proj[:, 0:1], NEG)
        rrn[:, rows] = rm.T

        @pl.when(tile == 0)
        def _():
            sm[...] = jnp.zeros_like(sm)
            mx[...] = jnp.full_like(mx, NEG)
            hs[...] = jnp.zeros_like(hs)
            fs[...] = jnp.zeros_like(fs)

        sm[...] += jnp.sum(y, axis=0, keepdims=True)
        mx[...] = jnp.maximum(mx[...],
                              jnp.max(jnp.where(am > 0, y, NEG), axis=0,
                                      keepdims=True))
        hs[...] += jnp.sum(h, axis=0, keepdims=True)
        fs[...] += jnp.sum(x, axis=0, keepdims=True)

        @pl.when(tile == GRID - 1)
        def _():
            disc_e = lax.rsqrt(jnp.float32(degc_base) - c)
            disa_e = lax.rsqrt(1.0 + c)
            xc = jnp.maximum((disc_e * disa_e) * hs[...]
                             + (disc_e * disc_e) * hc[...] + b_ref[...],
                             0.0) * c
            sag_epilogue(xc, rrn, wq_ref, brel_ref[0, 0], wnext_ref, c, kk,
                         cdst_ref, ycdst_ref)

    @pl.when((t >= GRID) & (t < 2 * GRID))
    def _():
        mid_phase(t - GRID, A, A, rr1, rr2, w2_ref, b2_ref, wq2_ref,
                  brel2_ref, w3_ref, c1r, sm1, mx1, float(K1 + 1), K2,
                  c2r, yc2, False)

    @pl.when((t >= 2 * GRID) & (t < 3 * GRID))
    def _():
        mid_phase(t - 2 * GRID, A, A, rr2, rr3, w3_ref, b3_ref, wq3_ref,
                  brel3_ref, w3_ref, c2r, sm2, mx2, float(K2 + 1), K3,
                  c3r, yc3, False)

    @pl.when(t >= 3 * GRID)
    def _():
        tile = t - 3 * GRID
        rows = pl.ds(pl.multiple_of(tile * TILE, TILE), TILE)
        xx = A[rows, :]
        s_row = sig[0, 0] + rr3[:, rows]
        alive_row = s_row >= thr[0, 0]
        g_row = jnp.where(alive_row, jnp.tanh(s_row), 0.0)
        am_row = jnp.where(alive_row, 1.0, 0.0)
        y = xx * g_row.T
        am = am_row.T

        @pl.when(tile == 0)
        def _():
            sm3[...] = jnp.zeros_like(sm3)
            mx3[...] = jnp.full_like(mx3, NEG)

        sm3[...] += jnp.sum(y, axis=0, keepdims=True)
        mx3[...] = jnp.maximum(mx3[...],
                               jnp.max(jnp.where(am > 0, y, NEG), axis=0,
                                       keepdims=True))

        @pl.when(tile == GRID - 1)
        def _():
            def merge(mx, sm, yc_ref, c_ref, k):
                cc = c_ref[0, 0]
                m = jnp.maximum(mx, jnp.where(cc > 0, yc_ref[...], NEG))
                return m, (sm + yc_ref[...]) / jnp.float32(k)

            m1, a1 = merge(mx1[...], sm1[...], yc1, c1r, K1)
            m2, a2 = merge(mx2[...], sm2[...], yc2, c2r, K2)
            m3, a3 = merge(mx3[...], sm3[...], yc3, c3r, K3)
            state = (jnp.dot((m1 + m2 + m3).astype(bf16), wlt_ref[...],
                             preferred_element_type=f32)
                     + jnp.dot((a1 + a2 + a3).astype(bf16), wlb_ref[...],
                               preferred_element_type=f32)
                     + bl_ref[...])
            sv = jnp.maximum(jnp.dot(state.astype(bf16), wv1_ref[...],
                                     preferred_element_type=f32)
                             + bv1_ref[...], 0.0)
            svb = sv.astype(bf16).astype(f32)
            v_ref[...] = jnp.reshape(jnp.sum(svb * wv2_ref[...])
                                     + bv2_ref[0, 0], (1, 1))


def kernel(obs, is_alive, W1, b1, Wrel1, brel1, Wroot1, W2, b2, Wrel2, brel2,
           Wroot2, W3, b3, Wrel3, brel3, Wroot3, Wl, bl, Wv1, bv1, Wv2, bv2):
    obs_p = jnp.pad(obs, ((0, NP - N), (0, 0))).astype(bf16)
    row = lambda w: w.reshape(1, -1).astype(f32)
    b1r, b2r, b3r, blr, bv1r = row(b1), row(b2), row(b3), row(bl), row(bv1)
    brel1r, brel2r, brel3r, bv2r = (row(brel1), row(brel2), row(brel3),
                                    row(bv2))
    wv2r = row(Wv2).astype(bf16).astype(f32)
    wq1 = jnp.concatenate([row(Wroot1), row(Wrel1)], axis=0).astype(bf16)
    wq2 = jnp.concatenate([row(Wroot2), row(Wrel2)], axis=0).astype(bf16)
    wq3 = jnp.concatenate([row(Wroot3), row(Wrel3)], axis=0).astype(bf16)
    w1b, w2b, w3b = W1.astype(bf16), W2.astype(bf16), W3.astype(bf16)
    wv1b = Wv1.astype(bf16)
    wlt, wlb = Wl[:512].astype(bf16), Wl[512:].astype(bf16)

    _C = lambda: pl.BlockSpec((1, 1), lambda t: (0, 0))
    _V = lambda n: pl.BlockSpec((1, n), lambda t: (0, 0))
    _F = lambda s: pl.BlockSpec(s, lambda t: tuple(0 for _ in s))

    v = pl.pallas_call(
        _mega_body,
        grid=(4 * GRID,),
        in_specs=[pl.BlockSpec((TILE, 256), lambda t: (jnp.where(t < GRID, t, 0), 0)),
                  _F((256, 512)), _V(512), _F((2, 512)), _C(),
                  _F((512, 512)), _V(512), _F((2, 512)), _C(),
                  _F((512, 512)), _V(512), _F((2, 512)), _C(),
                  _F((512, 512)), _F((512, 512)), _V(512),
                  _F((512, 512)), _V(512), _V(512), _C()],
        out_specs=_C(),
        out_shape=jax.ShapeDtypeStruct((1, 1), f32),
        scratch_shapes=[
            pltpu.VMEM((NP, 512), f32),
            pltpu.VMEM((1, NP), f32), pltpu.VMEM((1, NP), f32),
            pltpu.VMEM((1, NP), f32),
            pltpu.VMEM((1, 512), f32), pltpu.VMEM((1, 512), f32),
            pltpu.VMEM((1, 512), f32), pltpu.VMEM((1, 512), f32),
            pltpu.VMEM((1, 512), f32), pltpu.VMEM((1, 512), f32),
            pltpu.VMEM((1, 512), f32), pltpu.VMEM((1, 512), f32),
            pltpu.VMEM((1, 1), f32), pltpu.VMEM((1, 1), f32),
            pltpu.VMEM((1, 1), f32), pltpu.VMEM((1, 1), f32),
            pltpu.VMEM((1, 1), f32),
            pltpu.VMEM((1, 512), f32), pltpu.VMEM((1, 512), f32),
            pltpu.VMEM((1, 512), f32), pltpu.VMEM((1, 512), f32),
        ],
        compiler_params=pltpu.CompilerParams(
            dimension_semantics=("arbitrary",)),
    )(obs_p, w1b, b1r, wq1, brel1r, w2b, b2r, wq2, brel2r, w3b, b3r, wq3,
      brel3r, wlt, wlb, blr, wv1b, bv1r, wv2r, bv2r)

    return v.reshape(1)
